# baseline (speedup 1.0000x reference)
import functools

import jax
import jax.numpy as jnp
from jax import lax
from jax.experimental import pallas as pl
from jax.experimental.pallas import tpu as pltpu
from jax.experimental.pallas import tpu_sc as plsc

L = 16
BLK = 80
WACC = 136
NC, NS = 2, 16
NW = NC * NS

_PC = [32 * (p // 32) + 16 * (p % 2) + (p % 32) // 2 for p in range(128)]


def _mm2_body(x_ref, wl_ref, wr_ref, xl_ref, xr_ref):
    xv = x_ref[...]
    xl_ref[...] = jnp.dot(
        xv, wl_ref[...], preferred_element_type=jnp.float32
    ).astype(jnp.bfloat16)
    xr_ref[...] = jnp.dot(
        xv, wr_ref[...], preferred_element_type=jnp.float32
    ).astype(jnp.bfloat16)


def _mm2(x, wl, wr, blk_m):
    n, d = x.shape
    grid = (n // blk_m,)
    return pl.pallas_call(
        _mm2_body,
        grid=grid,
        in_specs=[
            pl.BlockSpec((blk_m, d), lambda i: (i, 0)),
            pl.BlockSpec((d, d), lambda i: (0, 0)),
            pl.BlockSpec((d, d), lambda i: (0, 0)),
        ],
        out_specs=[pl.BlockSpec((blk_m, d), lambda i: (i, 0))] * 2,
        out_shape=[jax.ShapeDtypeStruct((n, d), jnp.bfloat16)] * 2,
    )(x, wl, wr)


def _combine_mm2_body(a0_ref, a1_ref, b_ref, wl_ref, wr_ref, hl_ref, hr_ref):
    s = a0_ref[...] + a1_ref[...]
    h = s[:, :128] / s[:, 128:129] + b_ref[...]
    h = jnp.maximum(h, 0.0)
    hl_ref[...] = jnp.dot(
        h, wl_ref[...], preferred_element_type=jnp.float32
    ).astype(jnp.bfloat16)
    hr_ref[...] = jnp.dot(
        h, wr_ref[...], preferred_element_type=jnp.float32
    ).astype(jnp.bfloat16)


def _combine_mm2(a0, a1, b, wl, wr, blk_m):
    n = a0.shape[0]
    d = wl.shape[0]
    grid = (n // blk_m,)
    return pl.pallas_call(
        _combine_mm2_body,
        grid=grid,
        in_specs=[
            pl.BlockSpec((blk_m, WACC), lambda i: (i, 0)),
            pl.BlockSpec((blk_m, WACC), lambda i: (i, 0)),
            pl.BlockSpec((1, d), lambda i: (0, 0)),
            pl.BlockSpec((d, d), lambda i: (0, 0)),
            pl.BlockSpec((d, d), lambda i: (0, 0)),
        ],
        out_specs=[pl.BlockSpec((blk_m, d), lambda i: (i, 0))] * 2,
        out_shape=[jax.ShapeDtypeStruct((n, d), jnp.bfloat16)] * 2,
    )(a0, a1, b, wl, wr)


def _final_body(a0_ref, a1_ref, b_ref, out_ref):
    s = a0_ref[...] + a1_ref[...]
    out_ref[...] = s[:, :128] / s[:, 128:129] + b_ref[...]


def _final(a0, a1, b, blk_m):
    n = a0.shape[0]
    d = b.shape[1]
    grid = (n // blk_m,)
    return pl.pallas_call(
        _final_body,
        grid=grid,
        in_specs=[
            pl.BlockSpec((blk_m, WACC), lambda i: (i, 0)),
            pl.BlockSpec((blk_m, WACC), lambda i: (i, 0)),
            pl.BlockSpec((1, d), lambda i: (0, 0)),
        ],
        out_specs=pl.BlockSpec((blk_m, d), lambda i: (i, 0)),
        out_shape=jax.ShapeDtypeStruct((n, d), jnp.float32),
    )(a0, a1, b)


def _make_edge_pass(n_nodes, e_real, e_pad):
    epw = e_pad // NW
    nblk = epw // BLK
    npair = nblk // 2
    rows_sub = -(-n_nodes // NS // 8) * 8
    mesh = plsc.VectorSubcoreMesh(core_axis_name="c", subcore_axis_name="s")

    @functools.partial(
        pl.kernel,
        mesh=mesh,
        compiler_params=pltpu.CompilerParams(use_tc_tiling_on_sc=False,
                                             needs_layout_passes=False),
        out_type=jax.ShapeDtypeStruct((NC * n_nodes, WACC), jnp.float32),
        scratch_types=[
            pltpu.VMEM((128,), jnp.bfloat16),
            pltpu.VMEM((3, 4, BLK), jnp.int32),
            pltpu.VMEM((2, BLK, 128), jnp.bfloat16),
            pltpu.VMEM((2, BLK, 128), jnp.bfloat16),
            pltpu.VMEM((2, BLK, WACC), jnp.float32),
            pltpu.VMEM_SHARED((n_nodes, WACC), jnp.float32),
            pltpu.SemaphoreType.DMA((2,)),
            pltpu.SemaphoreType.DMA((2,)),
            pltpu.SemaphoreType.DMA((2,)),
        ],
    )
    def edge_pass(xl_hbm, xr_hbm, idx_hbm, att_hbm, out_hbm,
                  att_v, sd, xl_r, xr_r, w_r, acc, gsl, gsr, ssc):
        cid = lax.axis_index("c")
        sid = lax.axis_index("s")
        wid = sid * NC + cid

        pltpu.sync_copy(att_hbm, att_v)

        zeros = jnp.zeros((L,), jnp.float32)

        def zrow(i, _):
            for p in range(2):
                for k in range(8):
                    w_r[p, i, pl.ds(k * L, L)] = zeros
                w_r[p, i, pl.ds(WACC - L, L)] = zeros
            return 0
        lax.fori_loop(0, BLK, zrow, 0)

        row_start = pl.multiple_of(
            jnp.minimum(sid * rows_sub, n_nodes - rows_sub), 8)
        done = 0
        while done < rows_sub:
            sz = min(BLK, rows_sub - done)
            pltpu.sync_copy(w_r.at[0, pl.ds(0, sz)],
                            acc.at[pl.ds(row_start + done, sz)])
            done += sz
        plsc.subcore_barrier()

        lane = lax.iota(jnp.int32, L)
        lane0 = lane == 0
        perms = [jnp.bitwise_xor(lane, w) for w in (8, 4, 2, 1)]
        gd = lax.GatherDimensionNumbers(
            offset_dims=(), collapsed_slice_dims=(0,), start_index_map=(0,))

        def _allsum(p):
            for pm in perms:
                p = p + lax.gather(p, pm[:, None], gd, (1,),
                                   mode=lax.GatherScatterMode.PROMISE_IN_BOUNDS)
            return p

        att = [att_v[pl.ds(k * 2 * L, 2 * L)] for k in range(4)]
        idx_base = wid * npair * 4

        def start_gathers(b, jbuf, p):
            off = p * 2
            pltpu.async_copy(xl_hbm.at[sd.at[jbuf, off]], xl_r.at[p],
                             gsl.at[p])
            pltpu.async_copy(xr_hbm.at[sd.at[jbuf, off + 1]], xr_r.at[p],
                             gsr.at[p])

        pltpu.sync_copy(idx_hbm.at[pl.ds(idx_base, 4)], sd.at[0])
        start_gathers(0, 0, 0)
        start_gathers(1, 0, 1)

        def block_body(b, _):
            p = jnp.bitwise_and(b, 1)
            pair = b // 2
            jp = pair % 3
            jq = (pair + 1) % 3
            off = p * 2
            base = wid * epw + b * BLK

            @pl.when(b >= 2)
            def _():
                pltpu.make_async_copy(w_r.at[p], acc.at[sd.at[jp, off + 1]],
                                      ssc.at[p]).wait()
            pltpu.make_async_copy(xl_hbm.at[sd.at[jp, off]], xl_r.at[p],
                                  gsl.at[p]).wait()
            pltpu.make_async_copy(xr_hbm.at[sd.at[jp, off + 1]], xr_r.at[p],
                                  gsr.at[p]).wait()

            @plsc.parallel_loop(0, BLK, 1, unroll=8)
            def edge_body(r):
                xlv = [xl_r[p, r, pl.ds(k * 2 * L, 2 * L)] for k in range(4)]
                acc_e = zeros
                for k in range(4):
                    z = xlv[k] + xr_r[p, r, pl.ds(k * 2 * L, 2 * L)]
                    lr = jnp.maximum(z, z * jnp.bfloat16(0.2))
                    m = lr * att[k]
                    ma, mb = plsc.unpack(
                        m, format=plsc.PackFormat.INTERLEAVED)
                    acc_e = acc_e + ma + mb
                exv = jnp.exp(_allsum(acc_e))
                exv = jnp.where(base + r < e_real, exv,
                                jnp.zeros((L,), jnp.float32))
                w_r[p, r, pl.ds(WACC - L, L)] = jnp.where(lane == 8, exv,
                                                          0.0)
                for k in range(4):
                    xa, xb = plsc.unpack(
                        xlv[k], format=plsc.PackFormat.INTERLEAVED)
                    w_r[p, r, pl.ds(k * 2 * L, L)] = xa * exv
                    w_r[p, r, pl.ds(k * 2 * L + L, L)] = xb * exv

            @pl.when(jnp.logical_and(p == 0, b + 2 < nblk))
            def _():
                pltpu.sync_copy(
                    idx_hbm.at[pl.ds(idx_base + (pair + 1) * 4, 4)],
                    sd.at[jq])

            @pl.when(b + 2 < nblk)
            def _():
                start_gathers(b + 2, jq, p)

            pltpu.async_copy(w_r.at[p], acc.at[sd.at[jp, off + 1]],
                             ssc.at[p], add=True)
            return 0

        lax.fori_loop(0, nblk, block_body, 0)

        for p in range(2):
            pltpu.make_async_copy(w_r.at[p], acc.at[sd.at[0, p * 2 + 1]],
                                  ssc.at[p]).wait()
        plsc.subcore_barrier()

        done = 0
        while done < rows_sub:
            sz = min(BLK, rows_sub - done)
            row0 = pl.multiple_of(row_start + done, 8)
            pltpu.sync_copy(acc.at[pl.ds(row0, sz)],
                            out_hbm.at[pl.ds(cid * n_nodes + row0, sz)])
            done += sz

    return edge_pass


def kernel(x, edge_index, W1l, W1r, a1, b1, W2l, W2r, a2, b2):
    n, d = x.shape
    e_in = edge_index.shape[1]
    e_real = e_in + n
    grain = NW * 2 * BLK
    epw = ((e_real + grain - 1) // grain) * 2 * BLK
    e_pad = epw * NW
    npair = epw // (2 * BLK)

    ar = jnp.arange(n, dtype=jnp.int32)
    pad = jnp.zeros((e_pad - e_real,), jnp.int32)
    src = jnp.concatenate([edge_index[0].astype(jnp.int32), ar, pad])
    dst = jnp.concatenate([edge_index[1].astype(jnp.int32), ar, pad])
    s3 = src.reshape(NW, npair, 2, BLK)
    d3 = dst.reshape(NW, npair, 2, BLK)
    idx = jnp.stack([s3[:, :, 0], d3[:, :, 0], s3[:, :, 1], d3[:, :, 1]],
                    axis=2).reshape(NW * npair * 4, BLK)

    pc = jnp.asarray(_PC, jnp.int32)
    W1lp, W1rp = W1l[:, pc], W1r[:, pc]
    W2lp, W2rp = W2l[:, pc], W2r[:, pc]
    a1p = a1[pc].astype(jnp.bfloat16)
    a2p = a2[pc].astype(jnp.bfloat16)

    edge_pass = _make_edge_pass(n, e_real, e_pad)
    blk_m = 400

    xl1, xr1 = _mm2(x, W1lp, W1rp, blk_m)
    acc1 = edge_pass(xl1, xr1, idx, a1p)
    xl2, xr2 = _combine_mm2(acc1[:n], acc1[n:], b1.reshape(1, d), W2lp, W2rp,
                            blk_m)
    acc2 = edge_pass(xl2, xr2, idx, a2p)
    return _final(acc2[:n], acc2[n:], b2.reshape(1, d), blk_m)


# speedup vs baseline: 1.1597x; 1.1597x over previous
import functools

import jax
import jax.numpy as jnp
from jax import lax
from jax.experimental import pallas as pl
from jax.experimental.pallas import tpu as pltpu
from jax.experimental.pallas import tpu_sc as plsc

L = 16
BLK = 64
WACC = 136
NC, NS = 2, 16
NW = NC * NS

_PC = [32 * (p // 32) + 16 * (p % 2) + (p % 32) // 2 for p in range(128)]


def _mm2_body(x_ref, wl_ref, wr_ref, xl_ref, xr_ref):
    xv = x_ref[...]
    xl_ref[...] = jnp.dot(
        xv, wl_ref[...], preferred_element_type=jnp.float32
    ).astype(jnp.bfloat16)
    xr_ref[...] = jnp.dot(
        xv, wr_ref[...], preferred_element_type=jnp.float32
    ).astype(jnp.bfloat16)


def _mm2(x, wl, wr, blk_m):
    n, d = x.shape
    grid = (n // blk_m,)
    return pl.pallas_call(
        _mm2_body,
        grid=grid,
        in_specs=[
            pl.BlockSpec((blk_m, d), lambda i: (i, 0)),
            pl.BlockSpec((d, d), lambda i: (0, 0)),
            pl.BlockSpec((d, d), lambda i: (0, 0)),
        ],
        out_specs=[pl.BlockSpec((blk_m, d), lambda i: (i, 0))] * 2,
        out_shape=[jax.ShapeDtypeStruct((n, d), jnp.bfloat16)] * 2,
    )(x, wl, wr)


def _combine_mm2_body(a0_ref, a1_ref, e0_ref, e1_ref, b_ref, wl_ref, wr_ref,
                      hl_ref, hr_ref):
    s = a0_ref[...] + a1_ref[...]
    es = e0_ref[...] + e1_ref[...]
    h = s / es[:, 0:1] + b_ref[...]
    h = jnp.maximum(h, 0.0)
    hl_ref[...] = jnp.dot(
        h, wl_ref[...], preferred_element_type=jnp.float32
    ).astype(jnp.bfloat16)
    hr_ref[...] = jnp.dot(
        h, wr_ref[...], preferred_element_type=jnp.float32
    ).astype(jnp.bfloat16)


def _combine_mm2(acc, ex, b, wl, wr, blk_m):
    n = acc.shape[0] // 2
    d = wl.shape[0]
    grid = (n // blk_m,)
    nb = n // blk_m
    return pl.pallas_call(
        _combine_mm2_body,
        grid=grid,
        in_specs=[
            pl.BlockSpec((blk_m, d), lambda i: (i, 0)),
            pl.BlockSpec((blk_m, d), lambda i, nb=nb: (nb + i, 0)),
            pl.BlockSpec((blk_m, 8), lambda i: (i, 0)),
            pl.BlockSpec((blk_m, 8), lambda i, nb=nb: (nb + i, 0)),
            pl.BlockSpec((1, d), lambda i: (0, 0)),
            pl.BlockSpec((d, d), lambda i: (0, 0)),
            pl.BlockSpec((d, d), lambda i: (0, 0)),
        ],
        out_specs=[pl.BlockSpec((blk_m, d), lambda i: (i, 0))] * 2,
        out_shape=[jax.ShapeDtypeStruct((n, d), jnp.bfloat16)] * 2,
    )(acc, acc, ex, ex, b, wl, wr)


def _final_body(a0_ref, a1_ref, e0_ref, e1_ref, b_ref, out_ref):
    s = a0_ref[...] + a1_ref[...]
    es = e0_ref[...] + e1_ref[...]
    out_ref[...] = s / es[:, 0:1] + b_ref[...]


def _final(acc, ex, b, blk_m):
    n = acc.shape[0] // 2
    d = b.shape[1]
    grid = (n // blk_m,)
    nb = n // blk_m
    return pl.pallas_call(
        _final_body,
        grid=grid,
        in_specs=[
            pl.BlockSpec((blk_m, d), lambda i: (i, 0)),
            pl.BlockSpec((blk_m, d), lambda i, nb=nb: (nb + i, 0)),
            pl.BlockSpec((blk_m, 8), lambda i: (i, 0)),
            pl.BlockSpec((blk_m, 8), lambda i, nb=nb: (nb + i, 0)),
            pl.BlockSpec((1, d), lambda i: (0, 0)),
        ],
        out_specs=pl.BlockSpec((blk_m, d), lambda i: (i, 0)),
        out_shape=jax.ShapeDtypeStruct((n, d), jnp.float32),
    )(acc, acc, ex, ex, b)


def _make_edge_pass(n_nodes, e_real, e_pad):
    epw = e_pad // NW
    nblk = epw // BLK
    npair = nblk // 2
    rows_sub = -(-n_nodes // NS // 8) * 8
    mesh = plsc.VectorSubcoreMesh(core_axis_name="c", subcore_axis_name="s")

    @functools.partial(
        pl.kernel,
        mesh=mesh,
        compiler_params=pltpu.CompilerParams(use_tc_tiling_on_sc=False,
                                             needs_layout_passes=False),
        out_type=(jax.ShapeDtypeStruct((NC * n_nodes, 128), jnp.float32),
                  jax.ShapeDtypeStruct((NC * n_nodes, 8), jnp.float32)),
        scratch_types=[
            pltpu.VMEM((128,), jnp.bfloat16),
            pltpu.VMEM((3, 4, BLK), jnp.int32),
            pltpu.VMEM((2, BLK, 128), jnp.bfloat16),
            pltpu.VMEM((2, BLK, 128), jnp.bfloat16),
            pltpu.VMEM((2, BLK, WACC), jnp.float32),
            pltpu.VMEM_SHARED((n_nodes, WACC), jnp.float32),
            pltpu.SemaphoreType.DMA((2,)),
            pltpu.SemaphoreType.DMA((2,)),
            pltpu.SemaphoreType.DMA((2,)),
        ],
    )
    def edge_pass(xl_hbm, xr_hbm, idx_hbm, att_hbm, out_hbm, ex_hbm,
                  att_v, sd, xl_r, xr_r, w_r, acc, gsl, gsr, ssc):
        cid = lax.axis_index("c")
        sid = lax.axis_index("s")
        wid = sid * NC + cid

        pltpu.sync_copy(att_hbm, att_v)

        zeros = jnp.zeros((L,), jnp.float32)

        def zrow(i, _):
            for p in range(2):
                for k in range(8):
                    w_r[p, i, pl.ds(k * L, L)] = zeros
                w_r[p, i, pl.ds(WACC - L, L)] = zeros
            return 0
        lax.fori_loop(0, BLK, zrow, 0)

        row_start = pl.multiple_of(
            jnp.minimum(sid * rows_sub, n_nodes - rows_sub), 8)
        done = 0
        while done < rows_sub:
            sz = min(BLK, rows_sub - done)
            pltpu.sync_copy(w_r.at[0, pl.ds(0, sz)],
                            acc.at[pl.ds(row_start + done, sz)])
            done += sz
        plsc.subcore_barrier()

        lane = lax.iota(jnp.int32, L)
        lane0 = lane == 0
        perms = [jnp.bitwise_xor(lane, w) for w in (8, 4, 2, 1)]
        gd = lax.GatherDimensionNumbers(
            offset_dims=(), collapsed_slice_dims=(0,), start_index_map=(0,))

        def _allsum(p):
            for pm in perms:
                p = p + lax.gather(p, pm[:, None], gd, (1,),
                                   mode=lax.GatherScatterMode.PROMISE_IN_BOUNDS)
            return p

        att = [att_v[pl.ds(k * 2 * L, 2 * L)] for k in range(4)]
        idx_base = wid * npair * 4

        def start_gathers(b, jbuf, p):
            off = p * 2
            pltpu.async_copy(xl_hbm.at[sd.at[jbuf, off]], xl_r.at[p],
                             gsl.at[p])
            pltpu.async_copy(xr_hbm.at[sd.at[jbuf, off + 1]], xr_r.at[p],
                             gsr.at[p])

        pltpu.sync_copy(idx_hbm.at[pl.ds(idx_base, 4)], sd.at[0])
        start_gathers(0, 0, 0)
        start_gathers(1, 0, 1)

        def block_body(b, _):
            p = jnp.bitwise_and(b, 1)
            pair = b // 2
            jp = pair % 3
            jq = (pair + 1) % 3
            off = p * 2
            base = wid * epw + b * BLK

            @pl.when(b >= 2)
            def _():
                pltpu.make_async_copy(w_r.at[p], acc.at[sd.at[jp, off + 1]],
                                      ssc.at[p]).wait()
            pltpu.make_async_copy(xl_hbm.at[sd.at[jp, off]], xl_r.at[p],
                                  gsl.at[p]).wait()
            pltpu.make_async_copy(xr_hbm.at[sd.at[jp, off + 1]], xr_r.at[p],
                                  gsr.at[p]).wait()

            @plsc.parallel_loop(0, BLK, 1, unroll=8)
            def edge_body(r):
                xlv = [xl_r[p, r, pl.ds(k * 2 * L, 2 * L)] for k in range(4)]
                acc_e = zeros
                for k in range(4):
                    z = xlv[k] + xr_r[p, r, pl.ds(k * 2 * L, 2 * L)]
                    lr = jnp.maximum(z, z * jnp.bfloat16(0.2))
                    m = lr * att[k]
                    ma, mb = plsc.unpack(
                        m, format=plsc.PackFormat.INTERLEAVED)
                    acc_e = acc_e + ma + mb
                exv = jnp.exp(_allsum(acc_e))
                exv = jnp.where(base + r < e_real, exv,
                                jnp.zeros((L,), jnp.float32))
                w_r[p, r, pl.ds(WACC - L, L)] = jnp.where(lane == 8, exv,
                                                          0.0)
                for k in range(4):
                    xa, xb = plsc.unpack(
                        xlv[k], format=plsc.PackFormat.INTERLEAVED)
                    w_r[p, r, pl.ds(k * 2 * L, L)] = xa * exv
                    w_r[p, r, pl.ds(k * 2 * L + L, L)] = xb * exv

            @pl.when(jnp.logical_and(p == 0, b + 2 < nblk))
            def _():
                pltpu.sync_copy(
                    idx_hbm.at[pl.ds(idx_base + (pair + 1) * 4, 4)],
                    sd.at[jq])

            @pl.when(b + 2 < nblk)
            def _():
                start_gathers(b + 2, jq, p)

            pltpu.async_copy(w_r.at[p], acc.at[sd.at[jp, off + 1]],
                             ssc.at[p], add=True)
            return 0

        lax.fori_loop(0, nblk, block_body, 0)

        for p in range(2):
            pltpu.make_async_copy(w_r.at[p], acc.at[sd.at[0, p * 2 + 1]],
                                  ssc.at[p]).wait()
        plsc.subcore_barrier()

        done = 0
        while done < rows_sub:
            sz = min(BLK, rows_sub - done)
            row0 = pl.multiple_of(row_start + done, 8)
            orow = cid * n_nodes + row0
            pltpu.sync_copy(acc.at[pl.ds(row0, sz), pl.ds(0, 128)],
                            out_hbm.at[pl.ds(orow, sz)])
            pltpu.sync_copy(acc.at[pl.ds(row0, sz), pl.ds(128, 8)],
                            ex_hbm.at[pl.ds(orow, sz)])
            done += sz

    return edge_pass


def kernel(x, edge_index, W1l, W1r, a1, b1, W2l, W2r, a2, b2):
    n, d = x.shape
    e_in = edge_index.shape[1]
    e_real = e_in + n
    grain = NW * 2 * BLK
    epw = ((e_real + grain - 1) // grain) * 2 * BLK
    e_pad = epw * NW
    npair = epw // (2 * BLK)

    ar = jnp.arange(n, dtype=jnp.int32)
    tail = jnp.concatenate([ar, jnp.zeros((e_pad - e_real,), jnp.int32)])
    ed = jnp.concatenate(
        [edge_index.astype(jnp.int32),
         jnp.broadcast_to(tail, (2, e_pad - e_in))], axis=1)
    e4 = ed.reshape(2, NW, npair, 2, BLK)
    idx = jnp.stack([e4[0], e4[1]], axis=3).reshape(NW * npair * 4, BLK)

    def permc(w):
        return (w.reshape(d, 4, 2, L).swapaxes(2, 3).reshape(d, d))

    def permv(v):
        return v.reshape(4, 2, L).swapaxes(1, 2).reshape(d)

    W1lp, W1rp = permc(W1l), permc(W1r)
    W2lp, W2rp = permc(W2l), permc(W2r)
    a1p = permv(a1).astype(jnp.bfloat16)
    a2p = permv(a2).astype(jnp.bfloat16)

    edge_pass = _make_edge_pass(n, e_real, e_pad)
    blk_m = 2000

    xl1, xr1 = _mm2(x, W1lp, W1rp, blk_m)
    acc1, ex1 = edge_pass(xl1, xr1, idx, a1p)
    xl2, xr2 = _combine_mm2(acc1, ex1, b1.reshape(1, d), W2lp, W2rp, blk_m)
    acc2, ex2 = edge_pass(xl2, xr2, idx, a2p)
    return _final(acc2, ex2, b2.reshape(1, d), blk_m)
